# baseline (speedup 1.0000x reference)
import functools

import numpy as np

import jax
import jax.numpy as jnp
from jax import lax
from jax.experimental import pallas as pl
from jax.experimental.pallas import tpu as pltpu


def _tile(dim, candidates=(64, 32, 16, 8)):
    for t in candidates:
        if dim % t == 0:
            return t
    return dim


def _resize_matrix(n_in, n_out):
    o = np.arange(n_out, dtype=np.float64)
    src = (o + 0.5) * (n_in / n_out) - 0.5
    lo = np.floor(src).astype(np.int64)
    frac = (src - lo).astype(np.float32)
    lo_c = np.clip(lo, 0, n_in - 1)
    hi_c = np.clip(lo + 1, 0, n_in - 1)
    mat = np.zeros((n_out, n_in), dtype=np.float32)
    mat[np.arange(n_out), lo_c] += 1.0 - frac
    mat[np.arange(n_out), hi_c] += frac
    return mat


def _feat_kernel(wf_ref, wm_ref, x_ref, y_ref, s_ref, o_ref):
    c_in = x_ref.shape[1]
    h = x_ref.shape[2]
    w = x_ref.shape[3]
    hs = h // 3
    c_fea = wf_ref.shape[1]
    sel = s_ref[...]

    def feat(img_ref, w_off):
        x = img_ref[0].reshape(c_in, hs, 3, w)[:, :, 0, :]
        x = x.reshape(c_in * hs, w)
        xs = jnp.dot(x, sel, preferred_element_type=jnp.float32)
        acc = None
        for co in range(c_fea):
            f = wf_ref[0, co] * xs[0:hs]
            for ci in range(1, c_in):
                f = f + wf_ref[ci, co] * xs[ci * hs:(ci + 1) * hs]
            t = wm_ref[w_off + co, 0] * jnp.maximum(f, 0.0)
            acc = t if acc is None else acc + t
        return acc

    o_ref[0, 0] = feat(x_ref, 0)
    o_ref[1, 0] = feat(y_ref, c_fea)


def _features(x_img, y_img, w_fea, w_mat):
    n, c_in, h, w = x_img.shape
    hs, ws = h // 3, w // 3
    sel = np.zeros((w, ws), dtype=np.float32)
    sel[3 * np.arange(ws), np.arange(ws)] = 1.0
    return pl.pallas_call(
        _feat_kernel,
        out_shape=jax.ShapeDtypeStruct((2, n, hs, ws), jnp.float32),
        grid=(n,),
        in_specs=[
            pl.BlockSpec(memory_space=pltpu.MemorySpace.SMEM),
            pl.BlockSpec(memory_space=pltpu.MemorySpace.SMEM),
            pl.BlockSpec((1, c_in, h, w), lambda b: (b, 0, 0, 0)),
            pl.BlockSpec((1, c_in, h, w), lambda b: (b, 0, 0, 0)),
            pl.BlockSpec((w, ws), lambda b: (0, 0)),
        ],
        out_specs=pl.BlockSpec((2, 1, hs, ws), lambda b: (0, b, 0, 0)),
        compiler_params=pltpu.CompilerParams(
            dimension_semantics=("parallel",)),
    )(w_fea, w_mat, x_img, y_img, jnp.asarray(sel))


def _disp_kernel(px_ref, py_ref, uh_ref, uw_ref, o_ref, t1_ref, t2_ref):
    px = px_ref[0, 0]
    py = py_ref[0, 0]
    uh = uh_ref[...]
    uw = uw_ref[...]
    t3, hs = uh.shape
    ws, w3 = uw.shape
    d_low = t1_ref.shape[0]

    a = jnp.dot(uh, px, preferred_element_type=jnp.float32)
    b = jnp.dot(uh, py, preferred_element_type=jnp.float32)

    wpos = lax.broadcasted_iota(jnp.int32, (t3, ws), 1)
    for d in range(d_low):
        r = b if d == 0 else jnp.roll(b, d, axis=1)
        t1_ref[d] = jnp.where(wpos >= d, a + r, 0.0).astype(t1_ref.dtype)

    cd = 4 if d_low % 4 == 0 else 1
    mn = None
    for c in range(0, d_low, cd):
        xc = t1_ref[c:c + cd].reshape(cd * t3, ws)
        t2c = jnp.dot(xc, uw, preferred_element_type=jnp.float32)
        t2c = t2c.reshape(cd, t3, w3)
        t2_ref[c:c + cd] = t2c
        m = jnp.min(t2c, axis=0)
        mn = m if mn is None else jnp.minimum(mn, m)
    t2_ref[d_low] = t2_ref[d_low - 1]

    l3 = jnp.float32(1.4426950408889634 / 3.0)
    mnl3 = mn * l3
    qc0 = mnl3 - t2_ref[0] * l3

    def body(d, carry):
        num_a, num_b, den, qp, qc = carry
        qn = mnl3 - t2_ref[d + 1] * l3
        r = qc + qc
        e0 = jnp.exp2(qp + r)
        e1 = jnp.exp2(qc + r)
        e2 = jnp.exp2(r + qn)
        s = e0 + e1 + e2
        w = 3.0 * d.astype(jnp.float32) + 1.0
        num_a = num_a + w * s
        num_b = num_b + (e2 - e0)
        den = den + s
        return (num_a, num_b, den, qc, qn)

    z = jnp.zeros((t3, w3), jnp.float32)
    num_a, num_b, den, _, _ = lax.fori_loop(
        0, d_low, body, (z, z, z, qc0, qc0))
    o_ref[0] = (num_a + num_b) / den


def _disp(fea2, maxdisp):
    _, n, hs, ws = fea2.shape
    d_low = maxdisp // 3
    h3, w3 = hs * 3, ws * 3
    t3 = _tile(h3)
    uh = jnp.asarray(_resize_matrix(hs, h3))
    uw = jnp.asarray(_resize_matrix(ws, w3).T)
    return pl.pallas_call(
        _disp_kernel,
        out_shape=jax.ShapeDtypeStruct((n, h3, w3), jnp.float32),
        grid=(n, h3 // t3),
        in_specs=[
            pl.BlockSpec((1, 1, hs, ws), lambda bb, hh: (0, bb, 0, 0)),
            pl.BlockSpec((1, 1, hs, ws), lambda bb, hh: (1, bb, 0, 0)),
            pl.BlockSpec((t3, hs), lambda bb, hh: (hh, 0)),
            pl.BlockSpec((ws, w3), lambda bb, hh: (0, 0)),
        ],
        out_specs=pl.BlockSpec((1, t3, w3), lambda bb, hh: (bb, hh, 0)),
        scratch_shapes=[
            pltpu.VMEM((d_low, t3, ws), jnp.bfloat16),
            pltpu.VMEM((d_low + 1, t3, w3), jnp.float32),
        ],
        compiler_params=pltpu.CompilerParams(
            dimension_semantics=("parallel", "parallel")),
    )(fea2, fea2, uh, jnp.asarray(uw, jnp.bfloat16))


@functools.partial(jax.jit, static_argnames=("maxdisp",))
def _forward(x_img, y_img, w_fea, w_mat, *, maxdisp):
    fea2 = _features(x_img, y_img, w_fea, w_mat)
    return _disp(fea2, maxdisp)


def kernel(x_img, y_img, w_fea, w_mat):
    return _forward(x_img, y_img, w_fea, w_mat, maxdisp=192)


# speedup vs baseline: 1.9873x; 1.9873x over previous
import functools

import numpy as np

import jax
import jax.numpy as jnp
from jax import lax
from jax.experimental import pallas as pl
from jax.experimental.pallas import tpu as pltpu


def _tile(dim, candidates=(64, 32, 16, 8)):
    for t in candidates:
        if dim % t == 0:
            return t
    return dim


def _resize_matrix(n_in, n_out):
    o = np.arange(n_out, dtype=np.float64)
    src = (o + 0.5) * (n_in / n_out) - 0.5
    lo = np.floor(src).astype(np.int64)
    frac = (src - lo).astype(np.float32)
    lo_c = np.clip(lo, 0, n_in - 1)
    hi_c = np.clip(lo + 1, 0, n_in - 1)
    mat = np.zeros((n_out, n_in), dtype=np.float32)
    mat[np.arange(n_out), lo_c] += 1.0 - frac
    mat[np.arange(n_out), hi_c] += frac
    return mat


def _feat_kernel(wf_ref, wm_ref, x_ref, y_ref, s_ref, o_ref):
    c_in = x_ref.shape[1]
    h = x_ref.shape[2]
    w = x_ref.shape[3]
    hs = h // 3
    c_fea = wf_ref.shape[1]
    sel = s_ref[...]

    def feat(img_ref, w_off):
        x = img_ref[0].reshape(c_in, hs, 3, w)[:, :, 0, :]
        x = x.reshape(c_in * hs, w)
        xs = jnp.dot(x, sel, preferred_element_type=jnp.float32)
        acc = None
        for co in range(c_fea):
            f = wf_ref[0, co] * xs[0:hs]
            for ci in range(1, c_in):
                f = f + wf_ref[ci, co] * xs[ci * hs:(ci + 1) * hs]
            t = wm_ref[w_off + co, 0] * jnp.maximum(f, 0.0)
            acc = t if acc is None else acc + t
        return acc

    o_ref[0, 0] = feat(x_ref, 0)
    o_ref[1, 0] = feat(y_ref, c_fea)


def _features(x_img, y_img, w_fea, w_mat):
    n, c_in, h, w = x_img.shape
    hs, ws = h // 3, w // 3
    sel = np.zeros((w, ws), dtype=np.float32)
    sel[3 * np.arange(ws), np.arange(ws)] = 1.0
    return pl.pallas_call(
        _feat_kernel,
        out_shape=jax.ShapeDtypeStruct((2, n, hs, ws), jnp.float32),
        grid=(n,),
        in_specs=[
            pl.BlockSpec(memory_space=pltpu.MemorySpace.SMEM),
            pl.BlockSpec(memory_space=pltpu.MemorySpace.SMEM),
            pl.BlockSpec((1, c_in, h, w), lambda b: (b, 0, 0, 0)),
            pl.BlockSpec((1, c_in, h, w), lambda b: (b, 0, 0, 0)),
            pl.BlockSpec((w, ws), lambda b: (0, 0)),
        ],
        out_specs=pl.BlockSpec((2, 1, hs, ws), lambda b: (0, b, 0, 0)),
        compiler_params=pltpu.CompilerParams(
            dimension_semantics=("parallel",)),
    )(w_fea, w_mat, x_img, y_img, jnp.asarray(sel))


def _disp_kernel(px_ref, py_ref, uh_ref, uw_ref, o_ref, t1_ref, t2_ref):
    px = px_ref[0, 0]
    py = py_ref[0, 0]
    uh = uh_ref[...]
    uw = uw_ref[...]
    t3, hs = uh.shape
    ws, w3 = uw.shape
    d_low = t1_ref.shape[0]

    a = jnp.dot(uh, px, preferred_element_type=jnp.float32)
    b = jnp.dot(uh, py, preferred_element_type=jnp.float32)

    wpos = lax.broadcasted_iota(jnp.int32, (t3, ws), 1)
    for d in range(d_low):
        r = b if d == 0 else jnp.roll(b, d, axis=1)
        t1_ref[d] = jnp.where(wpos >= d, a + r, 0.0).astype(t1_ref.dtype)

    cd = 4 if d_low % 4 == 0 else 1
    mn = None
    for c in range(0, d_low, cd):
        xc = t1_ref[c:c + cd].reshape(cd * t3, ws)
        t2c = jnp.dot(xc, uw, preferred_element_type=jnp.float32)
        t2c = t2c.reshape(cd, t3, w3)
        t2_ref[c:c + cd] = t2c
        m = jnp.min(t2c, axis=0)
        mn = m if mn is None else jnp.minimum(mn, m)
    t2_ref[d_low] = t2_ref[d_low - 1]

    l3 = jnp.float32(1.4426950408889634 / 3.0)
    mnl3 = mn * l3
    qc = mnl3 - t2_ref[0] * l3
    qp = qc
    num_a = jnp.zeros((t3, w3), jnp.float32)
    num_b = jnp.zeros((t3, w3), jnp.float32)
    den = jnp.zeros((t3, w3), jnp.float32)
    for d in range(d_low):
        qn = mnl3 - t2_ref[d + 1] * l3
        r = qc + qc
        e0 = jnp.exp2(qp + r)
        e1 = jnp.exp2(qc + r)
        e2 = jnp.exp2(r + qn)
        s = e0 + e1 + e2
        num_a = num_a + (3.0 * d + 1.0) * s
        num_b = num_b + (e2 - e0)
        den = den + s
        qp = qc
        qc = qn
    o_ref[0] = (num_a + num_b) / den


def _disp(fea2, maxdisp):
    _, n, hs, ws = fea2.shape
    d_low = maxdisp // 3
    h3, w3 = hs * 3, ws * 3
    t3 = _tile(h3)
    uh = jnp.asarray(_resize_matrix(hs, h3))
    uw = jnp.asarray(_resize_matrix(ws, w3).T)
    return pl.pallas_call(
        _disp_kernel,
        out_shape=jax.ShapeDtypeStruct((n, h3, w3), jnp.float32),
        grid=(n, h3 // t3),
        in_specs=[
            pl.BlockSpec((1, 1, hs, ws), lambda bb, hh: (0, bb, 0, 0)),
            pl.BlockSpec((1, 1, hs, ws), lambda bb, hh: (1, bb, 0, 0)),
            pl.BlockSpec((t3, hs), lambda bb, hh: (hh, 0)),
            pl.BlockSpec((ws, w3), lambda bb, hh: (0, 0)),
        ],
        out_specs=pl.BlockSpec((1, t3, w3), lambda bb, hh: (bb, hh, 0)),
        scratch_shapes=[
            pltpu.VMEM((d_low, t3, ws), jnp.bfloat16),
            pltpu.VMEM((d_low + 1, t3, w3), jnp.float32),
        ],
        compiler_params=pltpu.CompilerParams(
            dimension_semantics=("parallel", "parallel")),
    )(fea2, fea2, uh, jnp.asarray(uw, jnp.bfloat16))


@functools.partial(jax.jit, static_argnames=("maxdisp",))
def _forward(x_img, y_img, w_fea, w_mat, *, maxdisp):
    fea2 = _features(x_img, y_img, w_fea, w_mat)
    return _disp(fea2, maxdisp)


def kernel(x_img, y_img, w_fea, w_mat):
    return _forward(x_img, y_img, w_fea, w_mat, maxdisp=192)


# speedup vs baseline: 2.4782x; 1.2470x over previous
import functools

import numpy as np

import jax
import jax.numpy as jnp
from jax import lax
from jax.experimental import pallas as pl
from jax.experimental.pallas import tpu as pltpu


def _tile(dim, candidates=(64, 32, 16, 8)):
    for t in candidates:
        if dim % t == 0:
            return t
    return dim


def _resize_matrix(n_in, n_out):
    o = np.arange(n_out, dtype=np.float64)
    src = (o + 0.5) * (n_in / n_out) - 0.5
    lo = np.floor(src).astype(np.int64)
    frac = (src - lo).astype(np.float32)
    lo_c = np.clip(lo, 0, n_in - 1)
    hi_c = np.clip(lo + 1, 0, n_in - 1)
    mat = np.zeros((n_out, n_in), dtype=np.float32)
    mat[np.arange(n_out), lo_c] += 1.0 - frac
    mat[np.arange(n_out), hi_c] += frac
    return mat


def _feat_kernel(wf_ref, wm_ref, x_ref, y_ref, s_ref, o_ref):
    c_in = x_ref.shape[1]
    h = x_ref.shape[2]
    w = x_ref.shape[3]
    hs = h // 3
    c_fea = wf_ref.shape[1]
    sel = s_ref[...]

    def feat(img_ref, w_off):
        x = img_ref[0].reshape(c_in, hs, 3, w)[:, :, 0, :]
        x = x.reshape(c_in * hs, w)
        xs = jnp.dot(x, sel, preferred_element_type=jnp.float32)
        acc = None
        for co in range(c_fea):
            f = wf_ref[0, co] * xs[0:hs]
            for ci in range(1, c_in):
                f = f + wf_ref[ci, co] * xs[ci * hs:(ci + 1) * hs]
            t = wm_ref[w_off + co, 0] * jnp.maximum(f, 0.0)
            acc = t if acc is None else acc + t
        return acc

    o_ref[0, 0] = feat(x_ref, 0)
    o_ref[1, 0] = feat(y_ref, c_fea)


def _features(x_img, y_img, w_fea, w_mat):
    n, c_in, h, w = x_img.shape
    hs, ws = h // 3, w // 3
    sel = np.zeros((w, ws), dtype=np.float32)
    sel[3 * np.arange(ws), np.arange(ws)] = 1.0
    return pl.pallas_call(
        _feat_kernel,
        out_shape=jax.ShapeDtypeStruct((2, n, hs, ws), jnp.float32),
        grid=(n,),
        in_specs=[
            pl.BlockSpec(memory_space=pltpu.MemorySpace.SMEM),
            pl.BlockSpec(memory_space=pltpu.MemorySpace.SMEM),
            pl.BlockSpec((1, c_in, h, w), lambda b: (b, 0, 0, 0)),
            pl.BlockSpec((1, c_in, h, w), lambda b: (b, 0, 0, 0)),
            pl.BlockSpec((w, ws), lambda b: (0, 0)),
        ],
        out_specs=pl.BlockSpec((2, 1, hs, ws), lambda b: (0, b, 0, 0)),
        compiler_params=pltpu.CompilerParams(
            dimension_semantics=("parallel",)),
    )(w_fea, w_mat, x_img, y_img, jnp.asarray(sel))


def _disp_kernel(px_ref, py_ref, uh_ref, uw_ref, o_ref, t1_ref, t2_ref):
    px = px_ref[0, 0]
    py = py_ref[0, 0]
    uh = uh_ref[...]
    uw = uw_ref[...]
    t3, hs = uh.shape
    ws, w3 = uw.shape
    d_low = t1_ref.shape[0]

    a = jnp.dot(uh, px, preferred_element_type=jnp.float32)
    b = jnp.dot(uh, py, preferred_element_type=jnp.float32)

    m_s = jnp.min(a) + jnp.min(b)
    a = a - m_s

    wpos = lax.broadcasted_iota(jnp.int32, (t3, ws), 1)
    for d in range(d_low):
        r = b if d == 0 else jnp.roll(b, d, axis=1)
        t1_ref[d] = jnp.where(wpos >= d, a + r, -m_s).astype(t1_ref.dtype)

    cd = 4 if d_low % 4 == 0 else 1
    for c in range(0, d_low, cd):
        xc = t1_ref[c:c + cd].reshape(cd * t3, ws)
        t2c = jnp.dot(xc, uw, preferred_element_type=jnp.float32)
        t2_ref[c:c + cd] = t2c.reshape(cd, t3, w3)
    t2_ref[d_low] = t2_ref[d_low - 1]

    qc = t2_ref[0]
    qp = qc
    num = jnp.zeros((t3, w3), jnp.float32)
    den = jnp.zeros((t3, w3), jnp.float32)
    for d in range(d_low):
        qn = t2_ref[d + 1]
        r = qc + qc
        e0 = jnp.exp2(qp + r)
        e1 = jnp.exp2(qc + r)
        e2 = jnp.exp2(r + qn)
        s = e0 + e1 + e2
        num = num + (3.0 * d + 1.0) * s + (e2 - e0)
        den = den + s
        qp = qc
        qc = qn
    o_ref[0] = num / den


def _disp(fea2, maxdisp):
    _, n, hs, ws = fea2.shape
    d_low = maxdisp // 3
    h3, w3 = hs * 3, ws * 3
    t3 = _tile(h3)
    uh = jnp.asarray(_resize_matrix(hs, h3))
    uw = jnp.asarray(_resize_matrix(ws, w3).T
                     * (-1.4426950408889634 / 3.0))
    return pl.pallas_call(
        _disp_kernel,
        out_shape=jax.ShapeDtypeStruct((n, h3, w3), jnp.float32),
        grid=(n, h3 // t3),
        in_specs=[
            pl.BlockSpec((1, 1, hs, ws), lambda bb, hh: (0, bb, 0, 0)),
            pl.BlockSpec((1, 1, hs, ws), lambda bb, hh: (1, bb, 0, 0)),
            pl.BlockSpec((t3, hs), lambda bb, hh: (hh, 0)),
            pl.BlockSpec((ws, w3), lambda bb, hh: (0, 0)),
        ],
        out_specs=pl.BlockSpec((1, t3, w3), lambda bb, hh: (bb, hh, 0)),
        scratch_shapes=[
            pltpu.VMEM((d_low, t3, ws), jnp.bfloat16),
            pltpu.VMEM((d_low + 1, t3, w3), jnp.float32),
        ],
        compiler_params=pltpu.CompilerParams(
            dimension_semantics=("parallel", "parallel")),
    )(fea2, fea2, uh, jnp.asarray(uw, jnp.bfloat16))


@functools.partial(jax.jit, static_argnames=("maxdisp",))
def _forward(x_img, y_img, w_fea, w_mat, *, maxdisp):
    fea2 = _features(x_img, y_img, w_fea, w_mat)
    return _disp(fea2, maxdisp)


def kernel(x_img, y_img, w_fea, w_mat):
    return _forward(x_img, y_img, w_fea, w_mat, maxdisp=192)


# speedup vs baseline: 2.6309x; 1.0616x over previous
import functools

import numpy as np

import jax
import jax.numpy as jnp
from jax import lax
from jax.experimental import pallas as pl
from jax.experimental.pallas import tpu as pltpu


def _tile(dim, candidates=(64, 32, 16, 8)):
    for t in candidates:
        if dim % t == 0:
            return t
    return dim


def _resize_matrix(n_in, n_out):
    o = np.arange(n_out, dtype=np.float64)
    src = (o + 0.5) * (n_in / n_out) - 0.5
    lo = np.floor(src).astype(np.int64)
    frac = (src - lo).astype(np.float32)
    lo_c = np.clip(lo, 0, n_in - 1)
    hi_c = np.clip(lo + 1, 0, n_in - 1)
    mat = np.zeros((n_out, n_in), dtype=np.float32)
    mat[np.arange(n_out), lo_c] += 1.0 - frac
    mat[np.arange(n_out), hi_c] += frac
    return mat


def _feat_kernel(wf_ref, wm_ref, x_ref, y_ref, s_ref, o_ref):
    c_in = x_ref.shape[1]
    h = x_ref.shape[2]
    w = x_ref.shape[3]
    hs = h // 3
    c_fea = wf_ref.shape[1]
    sel = s_ref[...]

    def feat(img_ref, w_off):
        x = img_ref[0].reshape(c_in, hs, 3, w)[:, :, 0, :]
        x = x.reshape(c_in * hs, w)
        xs = jnp.dot(x, sel, preferred_element_type=jnp.float32)
        acc = None
        for co in range(c_fea):
            f = wf_ref[0, co] * xs[0:hs]
            for ci in range(1, c_in):
                f = f + wf_ref[ci, co] * xs[ci * hs:(ci + 1) * hs]
            t = wm_ref[w_off + co, 0] * jnp.maximum(f, 0.0)
            acc = t if acc is None else acc + t
        return acc

    o_ref[0, 0] = feat(x_ref, 0)
    o_ref[1, 0] = feat(y_ref, c_fea)


def _features(x_img, y_img, w_fea, w_mat):
    n, c_in, h, w = x_img.shape
    hs, ws = h // 3, w // 3
    sel = np.zeros((w, ws), dtype=np.float32)
    sel[3 * np.arange(ws), np.arange(ws)] = 1.0
    return pl.pallas_call(
        _feat_kernel,
        out_shape=jax.ShapeDtypeStruct((2, n, hs, ws), jnp.float32),
        grid=(n, hs // 32),
        in_specs=[
            pl.BlockSpec(memory_space=pltpu.MemorySpace.SMEM),
            pl.BlockSpec(memory_space=pltpu.MemorySpace.SMEM),
            pl.BlockSpec((1, c_in, 96, w), lambda b, hh: (b, 0, hh, 0)),
            pl.BlockSpec((1, c_in, 96, w), lambda b, hh: (b, 0, hh, 0)),
            pl.BlockSpec((w, ws), lambda b, hh: (0, 0)),
        ],
        out_specs=pl.BlockSpec((2, 1, 32, ws), lambda b, hh: (0, b, hh, 0)),
        compiler_params=pltpu.CompilerParams(
            dimension_semantics=("parallel", "parallel")),
    )(w_fea, w_mat, x_img, y_img, jnp.asarray(sel))


def _disp_kernel(px_ref, py_ref, uh_ref, uw_ref, o_ref, t1_ref, t2_ref):
    px = px_ref[0, 0]
    py = py_ref[0, 0]
    uh = uh_ref[...]
    uw = uw_ref[...]
    t3, hs = uh.shape
    ws, w3 = uw.shape
    d_low = t1_ref.shape[0]

    a = jnp.dot(uh, px, preferred_element_type=jnp.float32)
    b = jnp.dot(uh, py, preferred_element_type=jnp.float32)

    m_s = jnp.min(a) + jnp.min(b)
    a = a - m_s

    wpos = lax.broadcasted_iota(jnp.int32, (t3, ws), 1)
    for d in range(d_low):
        r = b if d == 0 else jnp.roll(b, d, axis=1)
        t1_ref[d] = jnp.where(wpos >= d, a + r, -m_s).astype(t1_ref.dtype)

    cd = 4 if d_low % 4 == 0 else 1
    for c in range(0, d_low, cd):
        xc = t1_ref[c:c + cd].reshape(cd * t3, ws)
        t2c = jnp.dot(xc, uw, preferred_element_type=jnp.float32)
        t2_ref[c:c + cd] = t2c.reshape(cd, t3, w3)
    t2_ref[d_low] = t2_ref[d_low - 1]

    pc = jnp.exp2(t2_ref[0])
    pp = pc
    num = jnp.zeros((t3, w3), jnp.float32)
    den = jnp.zeros((t3, w3), jnp.float32)
    for d in range(d_low):
        pn = jnp.exp2(t2_ref[d + 1])
        p2 = pc * pc
        t = pp + pc + pn
        num = num + p2 * ((3.0 * d + 1.0) * t + (pn - pp))
        den = den + p2 * t
        pp = pc
        pc = pn
    o_ref[0] = num / den


def _disp(fea2, maxdisp):
    _, n, hs, ws = fea2.shape
    d_low = maxdisp // 3
    h3, w3 = hs * 3, ws * 3
    t3 = _tile(h3)
    uh = jnp.asarray(_resize_matrix(hs, h3))
    uw = jnp.asarray(_resize_matrix(ws, w3).T
                     * (-1.4426950408889634 / 3.0))
    return pl.pallas_call(
        _disp_kernel,
        out_shape=jax.ShapeDtypeStruct((n, h3, w3), jnp.float32),
        grid=(n, h3 // t3),
        in_specs=[
            pl.BlockSpec((1, 1, hs, ws), lambda bb, hh: (0, bb, 0, 0)),
            pl.BlockSpec((1, 1, hs, ws), lambda bb, hh: (1, bb, 0, 0)),
            pl.BlockSpec((t3, hs), lambda bb, hh: (hh, 0)),
            pl.BlockSpec((ws, w3), lambda bb, hh: (0, 0)),
        ],
        out_specs=pl.BlockSpec((1, t3, w3), lambda bb, hh: (bb, hh, 0)),
        scratch_shapes=[
            pltpu.VMEM((d_low, t3, ws), jnp.bfloat16),
            pltpu.VMEM((d_low + 1, t3, w3), jnp.float32),
        ],
        compiler_params=pltpu.CompilerParams(
            dimension_semantics=("parallel", "parallel")),
    )(fea2, fea2, uh, jnp.asarray(uw, jnp.bfloat16))


@functools.partial(jax.jit, static_argnames=("maxdisp",))
def _forward(x_img, y_img, w_fea, w_mat, *, maxdisp):
    fea2 = _features(x_img, y_img, w_fea, w_mat)
    return _disp(fea2, maxdisp)


def kernel(x_img, y_img, w_fea, w_mat):
    return _forward(x_img, y_img, w_fea, w_mat, maxdisp=192)


# speedup vs baseline: 2.6320x; 1.0004x over previous
import functools

import numpy as np

import jax
import jax.numpy as jnp
from jax import lax
from jax.experimental import pallas as pl
from jax.experimental.pallas import tpu as pltpu


def _tile(dim, candidates=(64, 32, 16, 8)):
    for t in candidates:
        if dim % t == 0:
            return t
    return dim


def _resize_matrix(n_in, n_out):
    o = np.arange(n_out, dtype=np.float64)
    src = (o + 0.5) * (n_in / n_out) - 0.5
    lo = np.floor(src).astype(np.int64)
    frac = (src - lo).astype(np.float32)
    lo_c = np.clip(lo, 0, n_in - 1)
    hi_c = np.clip(lo + 1, 0, n_in - 1)
    mat = np.zeros((n_out, n_in), dtype=np.float32)
    mat[np.arange(n_out), lo_c] += 1.0 - frac
    mat[np.arange(n_out), hi_c] += frac
    return mat


def _feat_kernel(wf_ref, wm_ref, x_ref, y_ref, s_ref, o_ref):
    c_in = x_ref.shape[1]
    h = x_ref.shape[2]
    w = x_ref.shape[3]
    hs = h // 3
    c_fea = wf_ref.shape[1]
    sel = s_ref[...]

    def feat(img_ref, w_off):
        x = img_ref[0].reshape(c_in, hs, 3, w)[:, :, 0, :]
        x = x.reshape(c_in * hs, w)
        xs = jnp.dot(x, sel, preferred_element_type=jnp.float32)
        acc = None
        for co in range(c_fea):
            f = wf_ref[0, co] * xs[0:hs]
            for ci in range(1, c_in):
                f = f + wf_ref[ci, co] * xs[ci * hs:(ci + 1) * hs]
            t = wm_ref[w_off + co, 0] * jnp.maximum(f, 0.0)
            acc = t if acc is None else acc + t
        return acc

    o_ref[0, 0] = feat(x_ref, 0)
    o_ref[1, 0] = feat(y_ref, c_fea)


def _features(x_img, y_img, w_fea, w_mat):
    n, c_in, h, w = x_img.shape
    hs, ws = h // 3, w // 3
    sel = np.zeros((w, ws), dtype=np.float32)
    sel[3 * np.arange(ws), np.arange(ws)] = 1.0
    return pl.pallas_call(
        _feat_kernel,
        out_shape=jax.ShapeDtypeStruct((2, n, hs, ws), jnp.float32),
        grid=(n, hs // 32),
        in_specs=[
            pl.BlockSpec(memory_space=pltpu.MemorySpace.SMEM),
            pl.BlockSpec(memory_space=pltpu.MemorySpace.SMEM),
            pl.BlockSpec((1, c_in, 96, w), lambda b, hh: (b, 0, hh, 0)),
            pl.BlockSpec((1, c_in, 96, w), lambda b, hh: (b, 0, hh, 0)),
            pl.BlockSpec((w, ws), lambda b, hh: (0, 0)),
        ],
        out_specs=pl.BlockSpec((2, 1, 32, ws), lambda b, hh: (0, b, hh, 0)),
        compiler_params=pltpu.CompilerParams(
            dimension_semantics=("parallel", "parallel")),
    )(w_fea, w_mat, x_img, y_img, jnp.asarray(sel))


def _disp_kernel(px_ref, py_ref, uh_ref, uw_ref, o_ref, *, d_low):
    px = px_ref[0, 0]
    py = py_ref[0, 0]
    uh = uh_ref[...]
    uw = uw_ref[...]
    t3, hs = uh.shape
    ws, w3 = uw.shape

    a = jnp.dot(uh, px, preferred_element_type=jnp.float32)
    b = jnp.dot(uh, py, preferred_element_type=jnp.float32)

    m_s = jnp.min(a) + jnp.min(b)
    a = a - m_s

    wpos = lax.broadcasted_iota(jnp.int32, (t3, ws), 1)
    neg_ms = -m_s
    cd = 4 if d_low % 4 == 0 else 1
    pp = pc = None
    num = jnp.zeros((t3, w3), jnp.float32)
    den = jnp.zeros((t3, w3), jnp.float32)

    def level(d, ppv, pcv, pnv, num, den):
        p2 = pcv * pcv
        t = ppv + pcv + pnv
        num = num + p2 * ((3.0 * d + 1.0) * t + (pnv - ppv))
        den = den + p2 * t
        return num, den

    for c in range(0, d_low, cd):
        rows = []
        for j in range(cd):
            d = c + j
            r = b if d == 0 else jnp.roll(b, d, axis=1)
            rows.append(jnp.where(wpos >= d, a + r, neg_ms).astype(jnp.bfloat16))
        xc = jnp.concatenate(rows, axis=0)
        q4 = jnp.dot(xc, uw, preferred_element_type=jnp.float32)
        q4 = q4.reshape(cd, t3, w3)
        for j in range(cd):
            pn = jnp.exp2(q4[j])
            if pc is None:
                pp = pc = pn
                continue
            num, den = level(c + j - 1, pp, pc, pn, num, den)
            pp = pc
            pc = pn
    num, den = level(d_low - 1, pp, pc, pc, num, den)
    o_ref[0] = num / den


def _disp(fea2, maxdisp):
    _, n, hs, ws = fea2.shape
    d_low = maxdisp // 3
    h3, w3 = hs * 3, ws * 3
    t3 = _tile(h3)
    uh = jnp.asarray(_resize_matrix(hs, h3))
    uw = jnp.asarray(_resize_matrix(ws, w3).T
                     * (-1.4426950408889634 / 3.0))
    return pl.pallas_call(
        functools.partial(_disp_kernel, d_low=d_low),
        out_shape=jax.ShapeDtypeStruct((n, h3, w3), jnp.float32),
        grid=(n, h3 // t3),
        in_specs=[
            pl.BlockSpec((1, 1, hs, ws), lambda bb, hh: (0, bb, 0, 0)),
            pl.BlockSpec((1, 1, hs, ws), lambda bb, hh: (1, bb, 0, 0)),
            pl.BlockSpec((t3, hs), lambda bb, hh: (hh, 0)),
            pl.BlockSpec((ws, w3), lambda bb, hh: (0, 0)),
        ],
        out_specs=pl.BlockSpec((1, t3, w3), lambda bb, hh: (bb, hh, 0)),
        compiler_params=pltpu.CompilerParams(
            dimension_semantics=("parallel", "parallel")),
    )(fea2, fea2, uh, jnp.asarray(uw, jnp.bfloat16))


@functools.partial(jax.jit, static_argnames=("maxdisp",))
def _forward(x_img, y_img, w_fea, w_mat, *, maxdisp):
    fea2 = _features(x_img, y_img, w_fea, w_mat)
    return _disp(fea2, maxdisp)


def kernel(x_img, y_img, w_fea, w_mat):
    return _forward(x_img, y_img, w_fea, w_mat, maxdisp=192)


# speedup vs baseline: 2.8438x; 1.0805x over previous
import functools

import numpy as np

import jax
import jax.numpy as jnp
from jax import lax
from jax.experimental import pallas as pl
from jax.experimental.pallas import tpu as pltpu


def _tile(dim, candidates=(64, 32, 16, 8)):
    for t in candidates:
        if dim % t == 0:
            return t
    return dim


def _resize_matrix(n_in, n_out):
    o = np.arange(n_out, dtype=np.float64)
    src = (o + 0.5) * (n_in / n_out) - 0.5
    lo = np.floor(src).astype(np.int64)
    frac = (src - lo).astype(np.float32)
    lo_c = np.clip(lo, 0, n_in - 1)
    hi_c = np.clip(lo + 1, 0, n_in - 1)
    mat = np.zeros((n_out, n_in), dtype=np.float32)
    mat[np.arange(n_out), lo_c] += 1.0 - frac
    mat[np.arange(n_out), hi_c] += frac
    return mat


def _feat_kernel(wf_ref, wm_ref, x_ref, y_ref, s_ref, o_ref):
    c_in = x_ref.shape[1]
    h = x_ref.shape[2]
    w = x_ref.shape[3]
    hs = h // 3
    c_fea = wf_ref.shape[1]
    sel = s_ref[...]

    def feat(img_ref, w_off):
        x = img_ref[0].reshape(c_in, hs, 3, w)[:, :, 0, :]
        x = x.reshape(c_in * hs, w)
        xs = jnp.dot(x, sel, preferred_element_type=jnp.float32)
        acc = None
        for co in range(c_fea):
            f = wf_ref[0, co] * xs[0:hs]
            for ci in range(1, c_in):
                f = f + wf_ref[ci, co] * xs[ci * hs:(ci + 1) * hs]
            t = wm_ref[w_off + co, 0] * jnp.maximum(f, 0.0)
            acc = t if acc is None else acc + t
        return acc

    o_ref[0, 0] = feat(x_ref, 0)
    o_ref[1, 0] = feat(y_ref, c_fea)


def _features(x_img, y_img, w_fea, w_mat):
    n, c_in, h, w = x_img.shape
    hs, ws = h // 3, w // 3
    sel = np.zeros((w, ws), dtype=np.float32)
    sel[3 * np.arange(ws), np.arange(ws)] = 1.0
    return pl.pallas_call(
        _feat_kernel,
        out_shape=jax.ShapeDtypeStruct((2, n, hs, ws), jnp.float32),
        grid=(n, hs // 32),
        in_specs=[
            pl.BlockSpec(memory_space=pltpu.MemorySpace.SMEM),
            pl.BlockSpec(memory_space=pltpu.MemorySpace.SMEM),
            pl.BlockSpec((1, c_in, 96, w), lambda b, hh: (b, 0, hh, 0)),
            pl.BlockSpec((1, c_in, 96, w), lambda b, hh: (b, 0, hh, 0)),
            pl.BlockSpec((w, ws), lambda b, hh: (0, 0)),
        ],
        out_specs=pl.BlockSpec((2, 1, 32, ws), lambda b, hh: (0, b, hh, 0)),
        compiler_params=pltpu.CompilerParams(
            dimension_semantics=("parallel", "parallel")),
    )(w_fea, w_mat, x_img, y_img, jnp.asarray(sel))


def _disp_kernel(px_ref, py_ref, uh_ref, uw_ref, o_ref, *, d_low):
    px = px_ref[0, 0]
    py = py_ref[0, 0]
    uh = uh_ref[...]
    uw = uw_ref[...]
    t3, hs = uh.shape
    ws, w3 = uw.shape

    a = jnp.dot(uh, px, preferred_element_type=jnp.float32)
    b = jnp.dot(uh, py, preferred_element_type=jnp.float32)

    m_s = jnp.min(a) + jnp.min(b)
    a = a - m_s

    wpos = lax.broadcasted_iota(jnp.int32, (t3, ws), 1)
    neg_ms = -m_s
    cd = 4 if d_low % 4 == 0 else 1
    pp = pc = None
    num = jnp.zeros((t3, w3), jnp.float32)
    den = jnp.zeros((t3, w3), jnp.float32)

    def level(d, ppv, pcv, pnv, num, den):
        p2 = pcv * pcv
        t = ppv + pcv + pnv
        num = num + p2 * ((3.0 * d + 1.0) * t + (pnv - ppv))
        den = den + p2 * t
        return num, den

    for c in range(0, d_low, cd):
        rows = []
        for j in range(cd):
            d = c + j
            r = b if d == 0 else jnp.roll(b, d, axis=1)
            rows.append(jnp.where(wpos >= d, a + r, neg_ms).astype(jnp.bfloat16))
        xc = jnp.concatenate(rows, axis=0)
        q4 = jnp.dot(xc, uw, preferred_element_type=jnp.float32)
        q4 = q4.reshape(cd, t3, w3)
        for j in range(cd):
            pn = jnp.exp2(q4[j])
            if pc is None:
                pp = pc = pn
                continue
            num, den = level(c + j - 1, pp, pc, pn, num, den)
            pp = pc
            pc = pn
    num, den = level(d_low - 1, pp, pc, pc, num, den)
    o_ref[0] = num / den


def _disp(fea2, maxdisp):
    _, n, hs, ws = fea2.shape
    d_low = maxdisp // 3
    h3, w3 = hs * 3, ws * 3
    t3 = _tile(h3, candidates=(128, 64, 32, 16, 8))
    uh = jnp.asarray(_resize_matrix(hs, h3))
    uw = jnp.asarray(_resize_matrix(ws, w3).T
                     * (-1.4426950408889634 / 3.0))
    return pl.pallas_call(
        functools.partial(_disp_kernel, d_low=d_low),
        out_shape=jax.ShapeDtypeStruct((n, h3, w3), jnp.float32),
        grid=(n, h3 // t3),
        in_specs=[
            pl.BlockSpec((1, 1, hs, ws), lambda bb, hh: (0, bb, 0, 0)),
            pl.BlockSpec((1, 1, hs, ws), lambda bb, hh: (1, bb, 0, 0)),
            pl.BlockSpec((t3, hs), lambda bb, hh: (hh, 0)),
            pl.BlockSpec((ws, w3), lambda bb, hh: (0, 0)),
        ],
        out_specs=pl.BlockSpec((1, t3, w3), lambda bb, hh: (bb, hh, 0)),
        compiler_params=pltpu.CompilerParams(
            dimension_semantics=("parallel", "parallel")),
    )(fea2, fea2, uh, jnp.asarray(uw, jnp.bfloat16))


@functools.partial(jax.jit, static_argnames=("maxdisp",))
def _forward(x_img, y_img, w_fea, w_mat, *, maxdisp):
    fea2 = _features(x_img, y_img, w_fea, w_mat)
    return _disp(fea2, maxdisp)


def kernel(x_img, y_img, w_fea, w_mat):
    return _forward(x_img, y_img, w_fea, w_mat, maxdisp=192)


# speedup vs baseline: 2.9500x; 1.0374x over previous
import functools

import numpy as np

import jax
import jax.numpy as jnp
from jax import lax
from jax.experimental import pallas as pl
from jax.experimental.pallas import tpu as pltpu


def _tile(dim, candidates=(64, 32, 16, 8)):
    for t in candidates:
        if dim % t == 0:
            return t
    return dim


def _resize_matrix(n_in, n_out):
    o = np.arange(n_out, dtype=np.float64)
    src = (o + 0.5) * (n_in / n_out) - 0.5
    lo = np.floor(src).astype(np.int64)
    frac = (src - lo).astype(np.float32)
    lo_c = np.clip(lo, 0, n_in - 1)
    hi_c = np.clip(lo + 1, 0, n_in - 1)
    mat = np.zeros((n_out, n_in), dtype=np.float32)
    mat[np.arange(n_out), lo_c] += 1.0 - frac
    mat[np.arange(n_out), hi_c] += frac
    return mat


def _feat_kernel(wf_ref, wm_ref, x_ref, y_ref, s_ref, o_ref):
    c_in = x_ref.shape[1]
    h = x_ref.shape[2]
    w = x_ref.shape[3]
    hs = h // 3
    c_fea = wf_ref.shape[1]
    sel = s_ref[...]

    def feat(img_ref, w_off):
        x = img_ref[0].reshape(c_in, hs, 3, w)[:, :, 0, :]
        x = x.reshape(c_in * hs, w)
        xs = jnp.dot(x, sel, preferred_element_type=jnp.float32)
        acc = None
        for co in range(c_fea):
            f = wf_ref[0, co] * xs[0:hs]
            for ci in range(1, c_in):
                f = f + wf_ref[ci, co] * xs[ci * hs:(ci + 1) * hs]
            t = wm_ref[w_off + co, 0] * jnp.maximum(f, 0.0)
            acc = t if acc is None else acc + t
        return acc

    o_ref[0, 0] = feat(x_ref, 0)
    o_ref[1, 0] = feat(y_ref, c_fea)


def _features(x_img, y_img, w_fea, w_mat):
    n, c_in, h, w = x_img.shape
    hs, ws = h // 3, w // 3
    sel = np.zeros((w, ws), dtype=np.float32)
    sel[3 * np.arange(ws), np.arange(ws)] = 1.0
    return pl.pallas_call(
        _feat_kernel,
        out_shape=jax.ShapeDtypeStruct((2, n, hs, ws), jnp.float32),
        grid=(n, hs // 64),
        in_specs=[
            pl.BlockSpec(memory_space=pltpu.MemorySpace.SMEM),
            pl.BlockSpec(memory_space=pltpu.MemorySpace.SMEM),
            pl.BlockSpec((1, c_in, 192, w), lambda b, hh: (b, 0, hh, 0)),
            pl.BlockSpec((1, c_in, 192, w), lambda b, hh: (b, 0, hh, 0)),
            pl.BlockSpec((w, ws), lambda b, hh: (0, 0)),
        ],
        out_specs=pl.BlockSpec((2, 1, 64, ws), lambda b, hh: (0, b, hh, 0)),
        compiler_params=pltpu.CompilerParams(
            dimension_semantics=("parallel", "parallel")),
    )(w_fea, w_mat, x_img, y_img, jnp.asarray(sel))


def _disp_kernel(px_ref, py_ref, uh_ref, uw_ref, o_ref, *, d_low):
    px = px_ref[0, 0]
    py = py_ref[0, 0]
    uh = uh_ref[...]
    uw = uw_ref[...]
    t3, hs = uh.shape
    ws, w3 = uw.shape

    a = jnp.dot(uh, px, preferred_element_type=jnp.float32)
    b = jnp.dot(uh, py, preferred_element_type=jnp.float32)

    m_s = jnp.min(a) + jnp.min(b)
    a = a - m_s

    wpos = lax.broadcasted_iota(jnp.int32, (t3, ws), 1)
    neg_ms = -m_s
    cd = 4 if d_low % 4 == 0 else 1
    pp = pc = None
    num = jnp.zeros((t3, w3), jnp.float32)
    den = jnp.zeros((t3, w3), jnp.float32)

    def level(d, ppv, pcv, pnv, num, den):
        p2 = pcv * pcv
        t = ppv + pcv + pnv
        num = num + p2 * ((3.0 * d + 1.0) * t + (pnv - ppv))
        den = den + p2 * t
        return num, den

    for c in range(0, d_low, cd):
        rows = []
        for j in range(cd):
            d = c + j
            r = b if d == 0 else jnp.roll(b, d, axis=1)
            rows.append(jnp.where(wpos >= d, a + r, neg_ms).astype(jnp.bfloat16))
        xc = jnp.concatenate(rows, axis=0)
        q4 = jnp.dot(xc, uw, preferred_element_type=jnp.float32)
        q4 = q4.reshape(cd, t3, w3)
        for j in range(cd):
            pn = jnp.exp2(q4[j])
            if pc is None:
                pp = pc = pn
                continue
            num, den = level(c + j - 1, pp, pc, pn, num, den)
            pp = pc
            pc = pn
    num, den = level(d_low - 1, pp, pc, pc, num, den)
    o_ref[0] = num / den


def _disp(fea2, maxdisp):
    _, n, hs, ws = fea2.shape
    d_low = maxdisp // 3
    h3, w3 = hs * 3, ws * 3
    t3 = _tile(h3, candidates=(192, 128, 64, 32, 16, 8))
    uh = jnp.asarray(_resize_matrix(hs, h3))
    uw = jnp.asarray(_resize_matrix(ws, w3).T
                     * (-1.4426950408889634 / 3.0))
    return pl.pallas_call(
        functools.partial(_disp_kernel, d_low=d_low),
        out_shape=jax.ShapeDtypeStruct((n, h3, w3), jnp.float32),
        grid=(n, h3 // t3),
        in_specs=[
            pl.BlockSpec((1, 1, hs, ws), lambda bb, hh: (0, bb, 0, 0)),
            pl.BlockSpec((1, 1, hs, ws), lambda bb, hh: (1, bb, 0, 0)),
            pl.BlockSpec((t3, hs), lambda bb, hh: (hh, 0)),
            pl.BlockSpec((ws, w3), lambda bb, hh: (0, 0)),
        ],
        out_specs=pl.BlockSpec((1, t3, w3), lambda bb, hh: (bb, hh, 0)),
        compiler_params=pltpu.CompilerParams(
            dimension_semantics=("parallel", "parallel")),
    )(fea2, fea2, uh, jnp.asarray(uw, jnp.bfloat16))


@functools.partial(jax.jit, static_argnames=("maxdisp",))
def _forward(x_img, y_img, w_fea, w_mat, *, maxdisp):
    fea2 = _features(x_img, y_img, w_fea, w_mat)
    return _disp(fea2, maxdisp)


def kernel(x_img, y_img, w_fea, w_mat):
    return _forward(x_img, y_img, w_fea, w_mat, maxdisp=192)
